# Initial kernel scaffold; baseline (speedup 1.0000x reference)
import jax
import jax.numpy as jnp
from jax.experimental import pallas as pl


def kernel(x, edge_index1, edge_index2, W_rel1, b_rel1, W_root1, W_rel2, b_rel2, W_root2, W_lin, b_lin, W_head, b_head):
    raise NotImplementedError("write your pallas kernel here")


# speedup vs baseline: 6.8226x; 6.8226x over previous
import functools

import jax
import jax.numpy as jnp
from jax import lax
from jax.experimental import pallas as pl
from jax.experimental.pallas import tpu as pltpu
from jax.experimental.pallas import tpu_sc as plsc

N0, N1, N2 = 50000, 10000, 2000
E1, E2 = 320000, 64000
F = 128
NC, NS = 2, 16
NW = NC * NS
K = 80
G = 25
ZR = 128


def _pad_rows(n):
    return -(-n // (NS * ZR)) * (NS * ZR)


@functools.lru_cache(maxsize=None)
def _make_segsum(E, N_out):
    EW = E // NW
    CH = EW // K
    GR = CH // G
    N_pad = _pad_rows(N_out)
    persub = N_pad // NS
    mesh = plsc.VectorSubcoreMesh(
        core_axis_name="c", subcore_axis_name="s",
        num_cores=NC, num_subcores=NS)

    @functools.partial(
        pl.kernel,
        out_type=jax.ShapeDtypeStruct((NC * N_pad, F), jnp.float32),
        mesh=mesh,
        scratch_types=[
            pltpu.VMEM((G, K), jnp.int32),
            pltpu.VMEM((G, K), jnp.int32),
            pltpu.VMEM((K, F), jnp.float32),
            pltpu.VMEM((8, F), jnp.float32),
            pltpu.VMEM_SHARED((N_pad, F), jnp.float32),
            pltpu.SemaphoreType.DMA,
        ],
    )
    def segsum(x_hbm, src_hbm, dst_hbm, out_hbm,
               src_v, dst_v, rows_v, zbuf, acc_sh, sem):
        cid = lax.axis_index("c")
        sid = lax.axis_index("s")
        wid = sid * NC + cid

        for i in range(8):
            for j in range(F // 16):
                zbuf[i, pl.ds(j * 16, 16)] = jnp.zeros((16,), jnp.float32)
        def zcopy(r, _):
            pltpu.sync_copy(zbuf, acc_sh.at[pl.ds(sid * persub + r * 8, 8)])
            return 0
        lax.fori_loop(0, persub // 8, zcopy, 0)
        plsc.subcore_barrier()

        def group(g, _):
            pltpu.sync_copy(src_hbm.at[wid, g], src_v)
            pltpu.sync_copy(dst_hbm.at[wid, g], dst_v)
            def body(c, _):
                pltpu.async_copy(x_hbm.at[src_v.at[c]], rows_v, sem).wait()
                pltpu.sync_copy(rows_v, acc_sh.at[dst_v.at[c]], add=True)
                return 0
            lax.fori_loop(0, G, body, 0)
            return 0
        lax.fori_loop(0, GR, group, 0)
        plsc.subcore_barrier()

        for r in range(persub // ZR):
            off = sid * persub + r * ZR
            pltpu.sync_copy(acc_sh.at[pl.ds(off, ZR)],
                            out_hbm.at[pl.ds(cid * N_pad + off, ZR)])

    return segsum


def _lin1_body(p_ref, xt_ref, wr_ref, wroot_ref, b_ref, o_ref):
    a = p_ref[0] + p_ref[1]
    acc = jnp.dot(a, wr_ref[...], preferred_element_type=jnp.float32)
    acc += jnp.dot(xt_ref[...], wroot_ref[...],
                   preferred_element_type=jnp.float32)
    o_ref[...] = jnp.maximum(acc + b_ref[...], 0.0)


def _head_body(p_ref, ht_ref, wr_ref, b2_ref, wroot_ref, wlin_ref, blin_ref,
               whead_ref, bhead_ref, o_ref):
    a = p_ref[0] + p_ref[1]
    h2 = jnp.dot(a, wr_ref[...], preferred_element_type=jnp.float32)
    h2 += jnp.dot(ht_ref[...], wroot_ref[...],
                  preferred_element_type=jnp.float32)
    h2 += b2_ref[...]
    t = jnp.dot(h2, wlin_ref[...], preferred_element_type=jnp.float32)
    t += blin_ref[...]
    o_ref[...] = jnp.dot(t, whead_ref[...],
                         preferred_element_type=jnp.float32) + bhead_ref[...]


def kernel(x, edge_index1, edge_index2, W_rel1, b_rel1, W_root1,
           W_rel2, b_rel2, W_root2, W_lin, b_lin, W_head, b_head):
    src1 = edge_index1[0].reshape(NW, E1 // (NW * G * K), G, K)
    dst1 = edge_index1[1].reshape(NW, E1 // (NW * G * K), G, K)
    p1 = _make_segsum(E1, N1)(x, src1, dst1)
    p1 = p1.reshape(NC, _pad_rows(N1), F)[:, :N1]

    R = 1000
    h = pl.pallas_call(
        _lin1_body,
        grid=(N1 // R,),
        in_specs=[
            pl.BlockSpec((NC, R, F), lambda i: (0, i, 0)),
            pl.BlockSpec((R, F), lambda i: (i, 0)),
            pl.BlockSpec((F, F), lambda i: (0, 0)),
            pl.BlockSpec((F, F), lambda i: (0, 0)),
            pl.BlockSpec((1, F), lambda i: (0, 0)),
        ],
        out_specs=pl.BlockSpec((R, F), lambda i: (i, 0)),
        out_shape=jax.ShapeDtypeStruct((N1, F), jnp.float32),
    )(p1, x[:N1], W_rel1, W_root1, b_rel1.reshape(1, F))

    src2 = edge_index2[0].reshape(NW, E2 // (NW * G * K), G, K)
    dst2 = edge_index2[1].reshape(NW, E2 // (NW * G * K), G, K)
    p2 = _make_segsum(E2, N2)(h, src2, dst2)
    p2 = p2.reshape(NC, _pad_rows(N2), F)[:, :N2]

    C = W_head.shape[1]
    W_head_p = jnp.zeros((F, F), jnp.float32).at[:, :C].set(W_head)
    b_head_p = jnp.zeros((1, F), jnp.float32).at[0, :C].set(b_head)
    full = lambda *s: pl.BlockSpec(s, lambda: tuple(0 for _ in s))
    out = pl.pallas_call(
        _head_body,
        in_specs=[
            full(NC, N2, F), full(N2, F), full(F, F), full(1, F),
            full(F, F), full(F, F), full(1, F), full(F, F), full(1, F),
        ],
        out_specs=full(N2, F),
        out_shape=jax.ShapeDtypeStruct((N2, F), jnp.float32),
    )(p2, h[:N2], W_rel2, b_rel2.reshape(1, F), W_root2,
      W_lin, b_lin.reshape(1, F), W_head_p, b_head_p)
    return out[:, :C]
